# Initial kernel scaffold; baseline (speedup 1.0000x reference)
import jax
import jax.numpy as jnp
from jax.experimental import pallas as pl


def kernel(entity_emb_sr, entity_emb_tg, relation_emb_sr, relation_emb_tg, head_sr, tail_sr, relation_sr, head_tg, tail_tg, relation_tg):
    raise NotImplementedError("write your pallas kernel here")


# speedup vs baseline: 2.2172x; 2.2172x over previous
import functools
import jax
import jax.numpy as jnp
import math
from jax import lax
from jax.experimental import pallas as pl
from jax.experimental.pallas import tpu as pltpu, tpu_sc as plsc

N_ENT = 10000
N_REL = 1000
N_TRI = 160000
D = 128

NW = 32
TPW = N_TRI // NW
TPW_PAD = 5120
CHUNK = 64
NCHUNKS = TPW_PAD // CHUNK

NB = 157
BLK_ROWS = 64
BLK_CELLS = BLK_ROWS * N_ENT
ACC_CELLS = 641024
ACC_PER_TILE = ACC_CELLS // 16
SIDE_SEG = 160
NSEG = 336
PAD_SEG = 159
REGION = TPW_PAD * 2 + NSEG * 8 + 128
SCALE = 3.0 * math.sqrt(D)

_mesh = plsc.VectorSubcoreMesh(core_axis_name="c", subcore_axis_name="s")


def _permute16(x, idx):
    dn = lax.GatherDimensionNumbers(
        offset_dims=(), collapsed_slice_dims=(0,), start_index_map=(0,))
    return lax.gather(x, idx[:, None], dn, (1,),
                      mode=lax.GatherScatterMode.PROMISE_IN_BOUNDS)


def _sum16(x, lane):
    for sh in (8, 4, 2, 1):
        x = x + _permute16(x, lane ^ sh)
    return x


def _sqrt16(x):
    x = jnp.maximum(x, jnp.float32(1e-30))
    i = lax.bitcast_convert_type(x, jnp.int32)
    i = jnp.int32(0x5F3759DF) - lax.shift_right_arithmetic(i, 1)
    y = lax.bitcast_convert_type(i, jnp.float32)
    half = jnp.float32(0.5) * x
    for _ in range(3):
        y = y * (jnp.float32(1.5) - half * y * y)
    return x * y


def _score_kernel(ent_sr, ent_tg, rel_sr, rel_tg,
                  h_sr, t_sr, r_sr, h_tg, t_tg, r_tg,
                  cells_out, scores_out, offs_out, cnts_out,
                  hidx, tidx, ridx, hrows, trows, rrows,
                  sums, within, bucket, score,
                  cells_stage, scores_stage, hist, cursor,
                  offs_stage, cnts_stage, sem_h, sem_t, sem_r):
    wid = lax.axis_index("s") * 2 + lax.axis_index("c")
    base = pl.multiple_of(wid * TPW, 8)
    lane = lax.iota(jnp.int32, 16)

    tables = ((ent_sr, rel_sr, h_sr, t_sr, r_sr),
              (ent_tg, rel_tg, h_tg, t_tg, r_tg))

    for side in range(2):
        ent, rel, h_all, t_all, r_all = tables[side]
        pltpu.sync_copy(h_all.at[pl.ds(base, TPW)], hidx.at[pl.ds(0, TPW)])
        pltpu.sync_copy(t_all.at[pl.ds(base, TPW)], tidx.at[pl.ds(0, TPW)])
        pltpu.sync_copy(r_all.at[pl.ds(base, TPW)], ridx.at[pl.ds(0, TPW)])

        for wi in range(8):
            o = TPW - 8 + wi * 16
            padv = lane + (wi * 16)
            keep = lane < 8 if wi == 0 else lane < 0
            hidx[pl.ds(o, 16)] = jnp.where(keep, hidx[pl.ds(o, 16)], padv)
            tidx[pl.ds(o, 16)] = jnp.where(keep, tidx[pl.ds(o, 16)], padv)
            ridx[pl.ds(o, 16)] = jnp.where(keep, ridx[pl.ds(o, 16)], padv)

        def chunk_body(k, _):
            off = k * CHUNK
            cp_h = pltpu.async_copy(ent.at[hidx.at[pl.ds(off, CHUNK)]], hrows, sem_h)
            cp_t = pltpu.async_copy(ent.at[tidx.at[pl.ds(off, CHUNK)]], trows, sem_t)
            cp_r = pltpu.async_copy(rel.at[ridx.at[pl.ds(off, CHUNK)]], rrows, sem_r)
            cp_h.wait()
            cp_t.wait()
            cp_r.wait()

            def grp_body(g, _):
                def tri_body(j, sv):
                    i = g * 16 + j
                    acc = jnp.zeros((16,), jnp.float32)
                    for jj in range(8):
                        hv = hrows[i, pl.ds(jj * 16, 16)]
                        tv = trows[i, pl.ds(jj * 16, 16)]
                        rv = rrows[i, pl.ds(jj * 16, 16)]
                        x = hv + rv - tv
                        acc = acc + x * x
                    s = _sum16(acc, lane)
                    return jnp.where(lane == j, s, sv)

                sv = lax.fori_loop(0, 16, tri_body, jnp.zeros((16,), jnp.float32))
                sums[pl.ds(off + g * 16, 16)] = sv
                return 0

            lax.fori_loop(0, CHUNK // 16, grp_body, 0)
            return 0

        lax.fori_loop(0, NCHUNKS, chunk_body, 0)

        def vec_body(v, _):
            o = v * 16
            s = sums[pl.ds(o, 16)]
            sc = jnp.float32(1.0) - _sqrt16(s) * jnp.float32(1.0 / SCALE)
            hv = hidx[pl.ds(o, 16)]
            tv = tidx[pl.ds(o, 16)]
            valid = (o + lane) < TPW
            within[side, pl.ds(o, 16)] = (hv & 63) * N_ENT + tv
            bucket[side, pl.ds(o, 16)] = jnp.where(
                valid, lax.shift_right_logical(hv, 6) + (side * SIDE_SEG), PAD_SEG)
            score[side, pl.ds(o, 16)] = sc
            return 0

        lax.fori_loop(0, TPW_PAD // 16, vec_body, 0)

    def hz_body(v, _):
        hist[pl.ds(v * 16, 16)] = jnp.zeros((16,), jnp.int32)
        return 0
    lax.fori_loop(0, NSEG // 16, hz_body, 0)

    def hist_body(v, _):
        for side in range(2):
            b = bucket[side, pl.ds(v * 16, 16)]
            cnt, is_last = plsc.scan_count(b)
            old = plsc.load_gather(hist, [b])
            plsc.store_scatter(hist, [b], old + cnt, mask=is_last)
        return 0
    lax.fori_loop(0, TPW_PAD // 16, hist_body, 0)

    def scan_body(v, off):
        c = hist[pl.ds(v * 16, 16)]
        padded = (c + 7) & ~7
        incl = plsc.cumsum(padded)
        vals = incl - padded + off
        cursor[pl.ds(v * 16, 16)] = vals
        offs_stage[pl.ds(v * 16, 16)] = vals
        cnts_stage[pl.ds(v * 16, 16)] = c
        return off + incl[15]
    lax.fori_loop(0, NSEG // 16, scan_body, jnp.int32(0))

    def scat_body(v, _):
        for side in range(2):
            b = bucket[side, pl.ds(v * 16, 16)]
            w = within[side, pl.ds(v * 16, 16)]
            s = score[side, pl.ds(v * 16, 16)]
            cnt, is_last = plsc.scan_count(b)
            old = plsc.load_gather(cursor, [b])
            pos = old + cnt - 1
            plsc.store_scatter(cells_stage, [pos], w)
            plsc.store_scatter(scores_stage, [pos], s)
            plsc.store_scatter(cursor, [b], old + cnt, mask=is_last)
        return 0
    lax.fori_loop(0, TPW_PAD // 16, scat_body, 0)

    pltpu.sync_copy(cells_stage, cells_out.at[wid])
    pltpu.sync_copy(scores_stage, scores_out.at[wid])
    pltpu.sync_copy(offs_stage, offs_out.at[wid])
    pltpu.sync_copy(cnts_stage, cnts_out.at[wid])


def _scatter_kernel(cells_hbm, scores_hbm, offs_hbm, cnts_hbm, out,
                    acc, zerobuf, offs_v, cnts_v,
                    cells_chunk, scores_chunk, rowbuf, sem_a, sem_b):
    side = lax.axis_index("c")
    tile = lax.axis_index("s")

    pltpu.sync_copy(offs_hbm, offs_v)
    pltpu.sync_copy(cnts_hbm, cnts_v)

    def z_body(v, _):
        zerobuf[pl.ds(v * 16, 16)] = jnp.zeros((16,), jnp.float32)
        return 0
    lax.fori_loop(0, ACC_PER_TILE // 16, z_body, 0)

    dump_idx = jnp.int32(BLK_CELLS) + lax.iota(jnp.int32, 16)

    def blk_body(b, _):
        pltpu.sync_copy(zerobuf, acc.at[pl.ds(pl.multiple_of(tile * ACC_PER_TILE, 8), ACC_PER_TILE)])
        plsc.subcore_barrier()

        seg = side * SIDE_SEG + b
        for w2 in range(2):
            w = tile + 16 * w2
            start = offs_v[w, pl.ds(seg, 16)][0] + w * REGION
            cnt = cnts_v[w, pl.ds(seg, 16)][0]
            nch = lax.shift_right_logical(cnt + 127, 7)

            def ch_body(k, _):
                o = pl.multiple_of(start + k * 128, 8)
                cp_a = pltpu.async_copy(
                    cells_hbm.at[pl.ds(o, 128)], cells_chunk.at[0], sem_a)
                cp_b = pltpu.async_copy(
                    scores_hbm.at[pl.ds(o, 128)], scores_chunk, sem_b)
                cp_a.wait()
                cp_b.wait()
                rem = cnt - k * 128
                for v in range(8):
                    lanes = lax.iota(jnp.int32, 16) + (16 * v)
                    m = lanes >= rem
                    cv = cells_chunk[0, pl.ds(v * 16, 16)]
                    sv = scores_chunk[pl.ds(v * 16, 16)]
                    cells_chunk[0, pl.ds(v * 16, 16)] = jnp.where(m, dump_idx, cv)
                    scores_chunk[pl.ds(v * 16, 16)] = jnp.where(m, jnp.float32(0.0), sv)
                pltpu.sync_copy(scores_chunk, acc.at[cells_chunk.at[0]], add=True)
                return 0

            lax.fori_loop(0, nch, ch_body, 0)
        plsc.subcore_barrier()

        nrows = jnp.where(b < NB - 1, 4, 1)

        def row_body(r, _):
            src0 = pl.multiple_of((tile * nrows + r) * N_ENT, 8)
            cell0 = pl.multiple_of(
                side * (N_ENT * N_ENT) + b * BLK_CELLS + (tile * nrows + r) * N_ENT, 8)
            pltpu.sync_copy(acc.at[pl.ds(src0, N_ENT)], rowbuf)
            pltpu.sync_copy(rowbuf, out.at[pl.ds(cell0, N_ENT)])
            return 0

        lax.fori_loop(0, nrows, row_body, 0)

        plsc.subcore_barrier()
        return 0

    lax.fori_loop(0, NB, blk_body, 0)


@jax.jit
def kernel(entity_emb_sr, entity_emb_tg, relation_emb_sr, relation_emb_tg,
           head_sr, tail_sr, relation_sr, head_tg, tail_tg, relation_tg):
    k1 = pl.kernel(
        _score_kernel,
        out_type=(
            jax.ShapeDtypeStruct((NW, REGION), jnp.int32),
            jax.ShapeDtypeStruct((NW, REGION), jnp.float32),
            jax.ShapeDtypeStruct((NW, NSEG), jnp.int32),
            jax.ShapeDtypeStruct((NW, NSEG), jnp.int32),
        ),
        mesh=_mesh,
        compiler_params=pltpu.CompilerParams(needs_layout_passes=False),
        scratch_types=[
            pltpu.VMEM((TPW_PAD,), jnp.int32),
            pltpu.VMEM((TPW_PAD,), jnp.int32),
            pltpu.VMEM((TPW_PAD,), jnp.int32),
            pltpu.VMEM((CHUNK, D), jnp.float32),
            pltpu.VMEM((CHUNK, D), jnp.float32),
            pltpu.VMEM((CHUNK, D), jnp.float32),
            pltpu.VMEM((TPW_PAD,), jnp.float32),
            pltpu.VMEM((2, TPW_PAD), jnp.int32),
            pltpu.VMEM((2, TPW_PAD), jnp.int32),
            pltpu.VMEM((2, TPW_PAD), jnp.float32),
            pltpu.VMEM((REGION,), jnp.int32),
            pltpu.VMEM((REGION,), jnp.float32),
            pltpu.VMEM((NSEG,), jnp.int32),
            pltpu.VMEM((NSEG,), jnp.int32),
            pltpu.VMEM((NSEG,), jnp.int32),
            pltpu.VMEM((NSEG,), jnp.int32),
            pltpu.SemaphoreType.DMA,
            pltpu.SemaphoreType.DMA,
            pltpu.SemaphoreType.DMA,
        ],
    )
    cells, scores, offs, cnts = k1(
        entity_emb_sr, entity_emb_tg, relation_emb_sr, relation_emb_tg,
        head_sr, tail_sr, relation_sr, head_tg, tail_tg, relation_tg)

    k3 = pl.kernel(
        _scatter_kernel,
        out_type=jax.ShapeDtypeStruct((2 * N_ENT * N_ENT,), jnp.float32),
        mesh=_mesh,
        compiler_params=pltpu.CompilerParams(needs_layout_passes=False),
        scratch_types=[
            pltpu.VMEM_SHARED((ACC_CELLS,), jnp.float32),
            pltpu.VMEM((ACC_PER_TILE,), jnp.float32),
            pltpu.VMEM((NW, NSEG), jnp.int32),
            pltpu.VMEM((NW, NSEG), jnp.int32),
            pltpu.VMEM((1, 128), jnp.int32),
            pltpu.VMEM((128,), jnp.float32),
            pltpu.VMEM((N_ENT,), jnp.float32),
            pltpu.SemaphoreType.DMA,
            pltpu.SemaphoreType.DMA,
        ],
    )
    cells_flat = cells.reshape((NW * REGION,))
    scores_flat = scores.reshape((NW * REGION,))
    out = k3(cells_flat, scores_flat, offs, cnts)
    return out.reshape((2, N_ENT, N_ENT))


# speedup vs baseline: 2.3843x; 1.0753x over previous
import jax
import jax.numpy as jnp
import math
from jax import lax
from jax.experimental import pallas as pl
from jax.experimental.pallas import tpu as pltpu, tpu_sc as plsc

N_ENT = 10000
N_REL = 1000
N_TRI = 160000
D = 128

NT = 16
TPW = N_TRI // NT
CHUNK = 16
NCHUNKS = TPW // CHUNK

NB = 157
BLK_ROWS = 64
BLK_CELLS = BLK_ROWS * N_ENT
ACC_CELLS = 640128
ACC_PER_TILE = ACC_CELLS // 16
NSEG = 176
REGION = TPW + NSEG * 8 + 128
ROWS_PER_DMA = 2
WCELLS = ROWS_PER_DMA * N_ENT
SCALE = 3.0 * math.sqrt(D)

_mesh = plsc.VectorSubcoreMesh(core_axis_name="c", subcore_axis_name="s")


def _permute16(x, idx):
    dn = lax.GatherDimensionNumbers(
        offset_dims=(), collapsed_slice_dims=(0,), start_index_map=(0,))
    return lax.gather(x, idx[:, None], dn, (1,),
                      mode=lax.GatherScatterMode.PROMISE_IN_BOUNDS)


def _sum16(x, lane):
    for sh in (8, 4, 2, 1):
        x = x + _permute16(x, lane ^ sh)
    return x


def _sqrt16(x):
    x = jnp.maximum(x, jnp.float32(1e-30))
    i = lax.bitcast_convert_type(x, jnp.int32)
    i = jnp.int32(0x5F3759DF) - lax.shift_right_arithmetic(i, 1)
    y = lax.bitcast_convert_type(i, jnp.float32)
    half = jnp.float32(0.5) * x
    for _ in range(3):
        y = y * (jnp.float32(1.5) - half * y * y)
    return x * y


def _fused_kernel(ent_sr, ent_tg, rel_sr, rel_tg,
                  h_sr, t_sr, r_sr, h_tg, t_tg, r_tg,
                  out,
                  acc,
                  hidx, tidx, ridx, hrows, trows, rrows,
                  score,
                  cells_stage, scores_stage, hist, cursor,
                  cbounce, sbounce, wbuf,
                  sem_h, sem_t, sem_r, sem_g, sem_w):
    side = lax.axis_index("c")
    tile = lax.axis_index("s")
    base = pl.multiple_of(tile * TPW, 8)
    lane = lax.iota(jnp.int32, 16)

    def p1(ent, rel, h_all, t_all, r_all):
        pltpu.sync_copy(h_all.at[pl.ds(base, TPW)], hidx.at[pl.ds(0, TPW)])
        pltpu.sync_copy(t_all.at[pl.ds(base, TPW)], tidx.at[pl.ds(0, TPW)])
        pltpu.sync_copy(r_all.at[pl.ds(base, TPW)], ridx.at[pl.ds(0, TPW)])

        def chunk_body(k, _):
            off = k * CHUNK
            cp_h = pltpu.async_copy(ent.at[hidx.at[pl.ds(off, CHUNK)]], hrows, sem_h)
            cp_t = pltpu.async_copy(ent.at[tidx.at[pl.ds(off, CHUNK)]], trows, sem_t)
            cp_r = pltpu.async_copy(rel.at[ridx.at[pl.ds(off, CHUNK)]], rrows, sem_r)
            cp_h.wait()
            cp_t.wait()
            cp_r.wait()

            def tri_body(j, sv):
                a = jnp.zeros((16,), jnp.float32)
                for jj in range(8):
                    hv = hrows[j, pl.ds(jj * 16, 16)]
                    tv = trows[j, pl.ds(jj * 16, 16)]
                    rv = rrows[j, pl.ds(jj * 16, 16)]
                    x = hv + rv - tv
                    a = a + x * x
                s = _sum16(a, lane)
                return jnp.where(lane == j, s, sv)

            sv = lax.fori_loop(0, 16, tri_body, jnp.zeros((16,), jnp.float32))
            score[pl.ds(off, 16)] = (
                jnp.float32(1.0) - _sqrt16(sv) * jnp.float32(1.0 / SCALE))
            return 0

        lax.fori_loop(0, NCHUNKS, chunk_body, 0)

    @pl.when(side == 0)
    def _():
        p1(ent_sr, rel_sr, h_sr, t_sr, r_sr)

    @pl.when(side == 1)
    def _():
        p1(ent_tg, rel_tg, h_tg, t_tg, r_tg)

    def hz_body(v, _):
        hist[pl.ds(v * 16, 16)] = jnp.zeros((16,), jnp.int32)
        return 0
    lax.fori_loop(0, NSEG // 16, hz_body, 0)

    def hist_body(v, _):
        b = lax.shift_right_logical(hidx[pl.ds(v * 16, 16)], 6)
        cnt, is_last = plsc.scan_count(b)
        old = plsc.load_gather(hist, [b])
        plsc.store_scatter(hist, [b], old + cnt, mask=is_last)
        return 0
    lax.fori_loop(0, TPW // 16, hist_body, 0)

    def scan_body(v, off):
        c = hist[pl.ds(v * 16, 16)]
        padded = (c + 7) & ~7
        incl = plsc.cumsum(padded)
        vals = incl - padded + off
        cursor[pl.ds(v * 16, 16)] = vals
        hist[pl.ds(v * 16, 16)] = vals
        return off + incl[15]
    lax.fori_loop(0, NSEG // 16, scan_body, jnp.int32(0))

    def sfill_body(v, _):
        cells_stage[pl.ds(v * 16, 16)] = jnp.full((16,), BLK_CELLS, jnp.int32) + lane
        scores_stage[pl.ds(v * 16, 16)] = jnp.zeros((16,), jnp.float32)
        return 0
    lax.fori_loop(0, REGION // 16, sfill_body, 0)

    def scat_body(v, _):
        o = v * 16
        hv = hidx[pl.ds(o, 16)]
        tv = tidx[pl.ds(o, 16)]
        b = lax.shift_right_logical(hv, 6)
        w = (hv & 63) * N_ENT + tv
        s = score[pl.ds(o, 16)]
        cnt, is_last = plsc.scan_count(b)
        old = plsc.load_gather(cursor, [b])
        pos = old + cnt - 1
        plsc.store_scatter(cells_stage, [pos], w)
        plsc.store_scatter(scores_stage, [pos], s)
        plsc.store_scatter(cursor, [b], old + cnt, mask=is_last)
        return 0
    lax.fori_loop(0, TPW // 16, scat_body, 0)

    def zf_body(v, _):
        score[pl.ds(v * 16, 16)] = jnp.zeros((16,), jnp.float32)
        return 0
    lax.fori_loop(0, TPW // 16, zf_body, 0)

    a0 = pl.multiple_of(tile * ACC_PER_TILE, 8)
    for zi in range(5):
        zl = ACC_PER_TILE - 4 * TPW if zi == 4 else TPW
        pltpu.sync_copy(score.at[pl.ds(0, zl)],
                        acc.at[pl.ds(pl.multiple_of(a0 + zi * TPW, 8), zl)])

    plsc.subcore_barrier()

    def blk_body(b, _):
        start = hist[pl.ds(b, 16)][0]
        cnt = cursor[pl.ds(b, 16)][0] - start
        nch = lax.shift_right_logical(cnt + 127, 7)

        def ch_body(k, _):
            o = pl.multiple_of(start + k * 128, 8)
            rem = cnt - k * 128
            dump = jnp.int32(BLK_CELLS) + lane
            for v in range(8):
                lv = lane + (16 * v)
                m = lv >= rem
                cv = cells_stage[pl.ds(o + v * 16, 16)]
                sv = scores_stage[pl.ds(o + v * 16, 16)]
                cbounce[0, pl.ds(v * 16, 16)] = jnp.where(m, dump, cv)
                sbounce[pl.ds(v * 16, 16)] = jnp.where(m, jnp.float32(0.0), sv)
            pltpu.sync_copy(sbounce, acc.at[cbounce.at[0]], add=True)
            return 0
        lax.fori_loop(0, nch, ch_body, 0)
        plsc.subcore_barrier()

        nrows = jnp.where(b < NB - 1, BLK_ROWS // 16, 1)

        def row_body(r, _):
            src0 = pl.multiple_of((tile * nrows + r) * N_ENT, 8)
            dst0 = pl.multiple_of(
                side * (N_ENT * N_ENT) + b * BLK_CELLS + (tile * nrows + r) * N_ENT, 8)
            pltpu.sync_copy(acc.at[pl.ds(src0, N_ENT)], wbuf)
            pltpu.sync_copy(wbuf, out.at[pl.ds(dst0, N_ENT)])
            return 0
        lax.fori_loop(0, nrows, row_body, 0)
        plsc.subcore_barrier()

        def zr_body(k, _):
            o = pl.multiple_of(start + k * 128, 8)
            for v in range(8):
                cbounce[0, pl.ds(v * 16, 16)] = cells_stage[pl.ds(o + v * 16, 16)]
            pltpu.sync_copy(score.at[pl.ds(0, 128)], acc.at[cbounce.at[0]])
            return 0
        lax.fori_loop(0, nch, zr_body, 0)
        plsc.subcore_barrier()
        return 0

    lax.fori_loop(0, NB, blk_body, 0)


@jax.jit
def kernel(entity_emb_sr, entity_emb_tg, relation_emb_sr, relation_emb_tg,
           head_sr, tail_sr, relation_sr, head_tg, tail_tg, relation_tg):
    k = pl.kernel(
        _fused_kernel,
        out_type=jax.ShapeDtypeStruct((2 * N_ENT * N_ENT,), jnp.float32),
        mesh=_mesh,
        compiler_params=pltpu.CompilerParams(needs_layout_passes=False),
        scratch_types=[
            pltpu.VMEM_SHARED((ACC_CELLS,), jnp.float32),
            pltpu.VMEM((TPW,), jnp.int32),
            pltpu.VMEM((TPW,), jnp.int32),
            pltpu.VMEM((TPW,), jnp.int32),
            pltpu.VMEM((CHUNK, D), jnp.float32),
            pltpu.VMEM((CHUNK, D), jnp.float32),
            pltpu.VMEM((CHUNK, D), jnp.float32),
            pltpu.VMEM((TPW,), jnp.float32),
            pltpu.VMEM((REGION,), jnp.int32),
            pltpu.VMEM((REGION,), jnp.float32),
            pltpu.VMEM((NSEG,), jnp.int32),
            pltpu.VMEM((NSEG,), jnp.int32),
            pltpu.VMEM((1, 128), jnp.int32),
            pltpu.VMEM((128,), jnp.float32),
            pltpu.VMEM((N_ENT,), jnp.float32),
            pltpu.SemaphoreType.DMA,
            pltpu.SemaphoreType.DMA,
            pltpu.SemaphoreType.DMA,
            pltpu.SemaphoreType.DMA,
            pltpu.SemaphoreType.DMA,
        ],
    )
    out = k(entity_emb_sr, entity_emb_tg, relation_emb_sr, relation_emb_tg,
            head_sr, tail_sr, relation_sr, head_tg, tail_tg, relation_tg)
    return out.reshape((2, N_ENT, N_ENT))


# speedup vs baseline: 2.5416x; 1.0660x over previous
import jax
import jax.numpy as jnp
import math
from jax import lax
from jax.experimental import pallas as pl
from jax.experimental.pallas import tpu as pltpu, tpu_sc as plsc

N_ENT = 10000
N_REL = 1000
N_TRI = 160000
D = 128

NT = 16
TPW = N_TRI // NT
CHUNK = 16
NCHUNKS = TPW // CHUNK

NB = 157
BLK_ROWS = 64
BLK_CELLS = BLK_ROWS * N_ENT
ACC_CELLS = 640128
ACC_PER_TILE = ACC_CELLS // 16
NSEG = 176
REGION = TPW + NSEG * 8 + 128
ROWS_PER_DMA = 2
WCELLS = ROWS_PER_DMA * N_ENT
SCALE = 3.0 * math.sqrt(D)

_mesh = plsc.VectorSubcoreMesh(core_axis_name="c", subcore_axis_name="s")


def _permute16(x, idx):
    dn = lax.GatherDimensionNumbers(
        offset_dims=(), collapsed_slice_dims=(0,), start_index_map=(0,))
    return lax.gather(x, idx[:, None], dn, (1,),
                      mode=lax.GatherScatterMode.PROMISE_IN_BOUNDS)


def _sum16(x, lane):
    for sh in (8, 4, 2, 1):
        x = x + _permute16(x, lane ^ sh)
    return x


def _sqrt16(x):
    x = jnp.maximum(x, jnp.float32(1e-30))
    i = lax.bitcast_convert_type(x, jnp.int32)
    i = jnp.int32(0x5F3759DF) - lax.shift_right_arithmetic(i, 1)
    y = lax.bitcast_convert_type(i, jnp.float32)
    half = jnp.float32(0.5) * x
    for _ in range(3):
        y = y * (jnp.float32(1.5) - half * y * y)
    return x * y


def _fused_kernel(ent_sr, ent_tg, rel_sr, rel_tg,
                  h_sr, t_sr, r_sr, h_tg, t_tg, r_tg,
                  out,
                  acc,
                  hidx, tidx, ridx, hrows, trows, rrows,
                  score,
                  cells_stage, scores_stage, hist, cursor,
                  cbounce, sbounce, wbuf,
                  sem_h, sem_t, sem_r, sem_g, sem_w):
    side = lax.axis_index("c")
    tile = lax.axis_index("s")
    base = pl.multiple_of(tile * TPW, 8)
    lane = lax.iota(jnp.int32, 16)

    def p1(ent, rel, h_all, t_all, r_all):
        pltpu.sync_copy(h_all.at[pl.ds(base, TPW)], hidx.at[pl.ds(0, TPW)])
        pltpu.sync_copy(t_all.at[pl.ds(base, TPW)], tidx.at[pl.ds(0, TPW)])
        pltpu.sync_copy(r_all.at[pl.ds(base, TPW)], ridx.at[pl.ds(0, TPW)])

        def chunk_body(k, _):
            off = k * CHUNK
            cp_h = pltpu.async_copy(ent.at[hidx.at[pl.ds(off, CHUNK)]], hrows, sem_h)
            cp_t = pltpu.async_copy(ent.at[tidx.at[pl.ds(off, CHUNK)]], trows, sem_t)
            cp_r = pltpu.async_copy(rel.at[ridx.at[pl.ds(off, CHUNK)]], rrows, sem_r)
            cp_h.wait()
            cp_t.wait()
            cp_r.wait()

            def tri_body(j, sv):
                a = jnp.zeros((16,), jnp.float32)
                for jj in range(8):
                    hv = hrows[j, pl.ds(jj * 16, 16)]
                    tv = trows[j, pl.ds(jj * 16, 16)]
                    rv = rrows[j, pl.ds(jj * 16, 16)]
                    x = hv + rv - tv
                    a = a + x * x
                s = _sum16(a, lane)
                return jnp.where(lane == j, s, sv)

            sv = lax.fori_loop(0, 16, tri_body, jnp.zeros((16,), jnp.float32))
            score[pl.ds(off, 16)] = (
                jnp.float32(1.0) - _sqrt16(sv) * jnp.float32(1.0 / SCALE))
            return 0

        lax.fori_loop(0, NCHUNKS, chunk_body, 0)

    @pl.when(side == 0)
    def _():
        p1(ent_sr, rel_sr, h_sr, t_sr, r_sr)

    @pl.when(side == 1)
    def _():
        p1(ent_tg, rel_tg, h_tg, t_tg, r_tg)

    def hz_body(v, _):
        hist[pl.ds(v * 16, 16)] = jnp.zeros((16,), jnp.int32)
        return 0
    lax.fori_loop(0, NSEG // 16, hz_body, 0)

    def hist_body(v, _):
        b = lax.shift_right_logical(hidx[pl.ds(v * 16, 16)], 6)
        cnt, is_last = plsc.scan_count(b)
        old = plsc.load_gather(hist, [b])
        plsc.store_scatter(hist, [b], old + cnt, mask=is_last)
        return 0
    lax.fori_loop(0, TPW // 16, hist_body, 0)

    def scan_body(v, off):
        c = hist[pl.ds(v * 16, 16)]
        padded = (c + 7) & ~7
        incl = plsc.cumsum(padded)
        vals = incl - padded + off
        cursor[pl.ds(v * 16, 16)] = vals
        hist[pl.ds(v * 16, 16)] = vals
        return off + incl[15]
    lax.fori_loop(0, NSEG // 16, scan_body, jnp.int32(0))

    def sfill_body(v, _):
        cells_stage[pl.ds(v * 16, 16)] = jnp.full((16,), BLK_CELLS, jnp.int32) + lane
        scores_stage[pl.ds(v * 16, 16)] = jnp.zeros((16,), jnp.float32)
        return 0
    lax.fori_loop(0, REGION // 16, sfill_body, 0)

    def scat_body(v, _):
        o = v * 16
        hv = hidx[pl.ds(o, 16)]
        tv = tidx[pl.ds(o, 16)]
        b = lax.shift_right_logical(hv, 6)
        w = (hv & 63) * N_ENT + tv
        s = score[pl.ds(o, 16)]
        cnt, is_last = plsc.scan_count(b)
        old = plsc.load_gather(cursor, [b])
        pos = old + cnt - 1
        plsc.store_scatter(cells_stage, [pos], w)
        plsc.store_scatter(scores_stage, [pos], s)
        plsc.store_scatter(cursor, [b], old + cnt, mask=is_last)
        return 0
    lax.fori_loop(0, TPW // 16, scat_body, 0)

    def zf_body(v, _):
        score[pl.ds(v * 16, 16)] = jnp.zeros((16,), jnp.float32)
        return 0
    lax.fori_loop(0, TPW // 16, zf_body, 0)

    a0 = pl.multiple_of(tile * ACC_PER_TILE, 8)
    for zi in range(5):
        zl = ACC_PER_TILE - 4 * TPW if zi == 4 else TPW
        pltpu.sync_copy(score.at[pl.ds(0, zl)],
                        acc.at[pl.ds(pl.multiple_of(a0 + zi * TPW, 8), zl)])

    plsc.subcore_barrier()

    def blk_body(b, _):
        start = hist[pl.ds(b, 16)][0]
        cnt = cursor[pl.ds(b, 16)][0] - start
        nch = lax.shift_right_logical(cnt + 127, 7)

        def ch_body(k, _):
            o = pl.multiple_of(start + k * 128, 8)
            rem = cnt - k * 128
            dump = jnp.int32(BLK_CELLS) + lane
            for v in range(8):
                lv = lane + (16 * v)
                m = lv >= rem
                cv = cells_stage[pl.ds(o + v * 16, 16)]
                sv = scores_stage[pl.ds(o + v * 16, 16)]
                cbounce[0, pl.ds(v * 16, 16)] = jnp.where(m, dump, cv)
                sbounce[pl.ds(v * 16, 16)] = jnp.where(m, jnp.float32(0.0), sv)
            pltpu.sync_copy(sbounce, acc.at[cbounce.at[0]], add=True)
            return 0
        lax.fori_loop(0, nch, ch_body, 0)
        plsc.subcore_barrier()

        def emit_halves(nh, src_base, dst_base):
            bufs = [wbuf.at[pl.ds(0, 5000)], wbuf.at[pl.ds(5000, 5000)]]
            cpg = pltpu.async_copy(acc.at[pl.ds(src_base, 5000)], bufs[0], sem_g)
            cpg.wait()
            prev_w = pltpu.async_copy(bufs[0], out.at[pl.ds(dst_base, 5000)], sem_w)
            for h in range(1, nh):
                buf = bufs[h % 2]
                cpg = pltpu.async_copy(
                    acc.at[pl.ds(pl.multiple_of(src_base + h * 5000, 8), 5000)],
                    buf, sem_g)
                cpg.wait()
                prev_w.wait()
                prev_w = pltpu.async_copy(
                    buf, out.at[pl.ds(pl.multiple_of(dst_base + h * 5000, 8), 5000)],
                    sem_w)
            prev_w.wait()

        @pl.when(b < NB - 1)
        def _():
            src0 = pl.multiple_of(tile * (4 * N_ENT), 8)
            dst0 = pl.multiple_of(
                side * (N_ENT * N_ENT) + b * BLK_CELLS + tile * (4 * N_ENT), 8)
            emit_halves(8, src0, dst0)

        @pl.when(b == NB - 1)
        def _():
            src0 = pl.multiple_of(tile * N_ENT, 8)
            dst0 = pl.multiple_of(
                side * (N_ENT * N_ENT) + (NB - 1) * BLK_CELLS + tile * N_ENT, 8)
            emit_halves(2, src0, dst0)

        plsc.subcore_barrier()

        def zr_body(k, _):
            o = pl.multiple_of(start + k * 128, 8)
            for v in range(8):
                cbounce[0, pl.ds(v * 16, 16)] = cells_stage[pl.ds(o + v * 16, 16)]
            pltpu.sync_copy(score.at[pl.ds(0, 128)], acc.at[cbounce.at[0]])
            return 0
        lax.fori_loop(0, nch, zr_body, 0)
        plsc.subcore_barrier()
        return 0

    lax.fori_loop(0, NB, blk_body, 0)


@jax.jit
def kernel(entity_emb_sr, entity_emb_tg, relation_emb_sr, relation_emb_tg,
           head_sr, tail_sr, relation_sr, head_tg, tail_tg, relation_tg):
    k = pl.kernel(
        _fused_kernel,
        out_type=jax.ShapeDtypeStruct((2 * N_ENT * N_ENT,), jnp.float32),
        mesh=_mesh,
        compiler_params=pltpu.CompilerParams(needs_layout_passes=False),
        scratch_types=[
            pltpu.VMEM_SHARED((ACC_CELLS,), jnp.float32),
            pltpu.VMEM((TPW,), jnp.int32),
            pltpu.VMEM((TPW,), jnp.int32),
            pltpu.VMEM((TPW,), jnp.int32),
            pltpu.VMEM((CHUNK, D), jnp.float32),
            pltpu.VMEM((CHUNK, D), jnp.float32),
            pltpu.VMEM((CHUNK, D), jnp.float32),
            pltpu.VMEM((TPW,), jnp.float32),
            pltpu.VMEM((REGION,), jnp.int32),
            pltpu.VMEM((REGION,), jnp.float32),
            pltpu.VMEM((NSEG,), jnp.int32),
            pltpu.VMEM((NSEG,), jnp.int32),
            pltpu.VMEM((1, 128), jnp.int32),
            pltpu.VMEM((128,), jnp.float32),
            pltpu.VMEM((N_ENT,), jnp.float32),
            pltpu.SemaphoreType.DMA,
            pltpu.SemaphoreType.DMA,
            pltpu.SemaphoreType.DMA,
            pltpu.SemaphoreType.DMA,
            pltpu.SemaphoreType.DMA,
        ],
    )
    out = k(entity_emb_sr, entity_emb_tg, relation_emb_sr, relation_emb_tg,
            head_sr, tail_sr, relation_sr, head_tg, tail_tg, relation_tg)
    return out.reshape((2, N_ENT, N_ENT))


# speedup vs baseline: 2.5478x; 1.0024x over previous
import jax
import jax.numpy as jnp
import math
from jax import lax
from jax.experimental import pallas as pl
from jax.experimental.pallas import tpu as pltpu, tpu_sc as plsc

N_ENT = 10000
N_REL = 1000
N_TRI = 160000
D = 128

NT = 16
TPW = N_TRI // NT
CHUNK = 16
NCHUNKS = TPW // CHUNK

NB = 157
BLK_ROWS = 64
BLK_CELLS = BLK_ROWS * N_ENT
ACC_CELLS = 640128
ACC_PER_TILE = ACC_CELLS // 16
NSEG = 176
REGION = TPW + NSEG * 8 + 128
ROWS_PER_DMA = 2
WCELLS = ROWS_PER_DMA * N_ENT
SCALE = 3.0 * math.sqrt(D)

_mesh = plsc.VectorSubcoreMesh(core_axis_name="c", subcore_axis_name="s")


def _permute16(x, idx):
    dn = lax.GatherDimensionNumbers(
        offset_dims=(), collapsed_slice_dims=(0,), start_index_map=(0,))
    return lax.gather(x, idx[:, None], dn, (1,),
                      mode=lax.GatherScatterMode.PROMISE_IN_BOUNDS)


def _sum16(x, lane):
    for sh in (8, 4, 2, 1):
        x = x + _permute16(x, lane ^ sh)
    return x


def _sqrt16(x):
    x = jnp.maximum(x, jnp.float32(1e-30))
    i = lax.bitcast_convert_type(x, jnp.int32)
    i = jnp.int32(0x5F3759DF) - lax.shift_right_arithmetic(i, 1)
    y = lax.bitcast_convert_type(i, jnp.float32)
    half = jnp.float32(0.5) * x
    for _ in range(3):
        y = y * (jnp.float32(1.5) - half * y * y)
    return x * y


def _fused_kernel(ent_sr, ent_tg, rel_sr, rel_tg,
                  h_sr, t_sr, r_sr, h_tg, t_tg, r_tg,
                  out,
                  acc,
                  hidx, tidx, ridx, hrows, trows, rrows,
                  score,
                  cells_stage, scores_stage, hist, cursor,
                  cbounce, sbounce, wbuf,
                  sem_h, sem_t, sem_r, sem_g, sem_w):
    side = lax.axis_index("c")
    tile = lax.axis_index("s")
    base = pl.multiple_of(tile * TPW, 8)
    lane = lax.iota(jnp.int32, 16)

    def p1(ent, rel, h_all, t_all, r_all):
        pltpu.sync_copy(h_all.at[pl.ds(base, TPW)], hidx.at[pl.ds(0, TPW)])
        pltpu.sync_copy(t_all.at[pl.ds(base, TPW)], tidx.at[pl.ds(0, TPW)])
        pltpu.sync_copy(r_all.at[pl.ds(base, TPW)], ridx.at[pl.ds(0, TPW)])

        def chunk_body(k, _):
            off = k * CHUNK
            cp_h = pltpu.async_copy(ent.at[hidx.at[pl.ds(off, CHUNK)]], hrows, sem_h)
            cp_t = pltpu.async_copy(ent.at[tidx.at[pl.ds(off, CHUNK)]], trows, sem_t)
            cp_r = pltpu.async_copy(rel.at[ridx.at[pl.ds(off, CHUNK)]], rrows, sem_r)
            cp_h.wait()
            cp_t.wait()
            cp_r.wait()

            def tri_body(j, sv):
                a = jnp.zeros((16,), jnp.float32)
                for jj in range(8):
                    hv = hrows[j, pl.ds(jj * 16, 16)]
                    tv = trows[j, pl.ds(jj * 16, 16)]
                    rv = rrows[j, pl.ds(jj * 16, 16)]
                    x = hv + rv - tv
                    a = a + x * x
                s = _sum16(a, lane)
                return jnp.where(lane == j, s, sv)

            sv = lax.fori_loop(0, 16, tri_body, jnp.zeros((16,), jnp.float32))
            score[pl.ds(off, 16)] = (
                jnp.float32(1.0) - _sqrt16(sv) * jnp.float32(1.0 / SCALE))
            return 0

        lax.fori_loop(0, NCHUNKS, chunk_body, 0)

    @pl.when(side == 0)
    def _():
        p1(ent_sr, rel_sr, h_sr, t_sr, r_sr)

    @pl.when(side == 1)
    def _():
        p1(ent_tg, rel_tg, h_tg, t_tg, r_tg)

    def hz_body(v, _):
        hist[pl.ds(v * 16, 16)] = jnp.zeros((16,), jnp.int32)
        return 0
    lax.fori_loop(0, NSEG // 16, hz_body, 0)

    def hist_body(v, _):
        b = lax.shift_right_logical(hidx[pl.ds(v * 16, 16)], 6)
        cnt, is_last = plsc.scan_count(b)
        old = plsc.load_gather(hist, [b])
        plsc.store_scatter(hist, [b], old + cnt, mask=is_last)
        return 0
    lax.fori_loop(0, TPW // 16, hist_body, 0)

    def scan_body(v, off):
        c = hist[pl.ds(v * 16, 16)]
        padded = (c + 7) & ~7
        incl = plsc.cumsum(padded)
        vals = incl - padded + off
        cursor[pl.ds(v * 16, 16)] = vals
        hist[pl.ds(v * 16, 16)] = vals
        return off + incl[15]
    lax.fori_loop(0, NSEG // 16, scan_body, jnp.int32(0))

    def sfill_body(v, _):
        cells_stage[pl.ds(v * 16, 16)] = jnp.full((16,), BLK_CELLS, jnp.int32) + lane
        scores_stage[pl.ds(v * 16, 16)] = jnp.zeros((16,), jnp.float32)
        return 0
    lax.fori_loop(0, REGION // 16, sfill_body, 0)

    def scat_body(v, _):
        o = v * 16
        hv = hidx[pl.ds(o, 16)]
        tv = tidx[pl.ds(o, 16)]
        b = lax.shift_right_logical(hv, 6)
        w = (hv & 63) * N_ENT + tv
        s = score[pl.ds(o, 16)]
        cnt, is_last = plsc.scan_count(b)
        old = plsc.load_gather(cursor, [b])
        pos = old + cnt - 1
        plsc.store_scatter(cells_stage, [pos], w)
        plsc.store_scatter(scores_stage, [pos], s)
        plsc.store_scatter(cursor, [b], old + cnt, mask=is_last)
        return 0
    lax.fori_loop(0, TPW // 16, scat_body, 0)

    def zf_body(v, _):
        score[pl.ds(v * 16, 16)] = jnp.zeros((16,), jnp.float32)
        return 0
    lax.fori_loop(0, TPW // 16, zf_body, 0)

    a0 = pl.multiple_of(tile * ACC_PER_TILE, 8)
    for zi in range(5):
        zl = ACC_PER_TILE - 4 * TPW if zi == 4 else TPW
        pltpu.sync_copy(score.at[pl.ds(0, zl)],
                        acc.at[pl.ds(pl.multiple_of(a0 + zi * TPW, 8), zl)])

    plsc.subcore_barrier()

    def blk_body(b, _):
        start = hist[pl.ds(b, 16)][0]
        cnt = cursor[pl.ds(b, 16)][0] - start
        nch = lax.shift_right_logical(cnt + 127, 7)

        def ch_body(k, _):
            o = pl.multiple_of(start + k * 128, 8)
            rem = cnt - k * 128
            dump = jnp.int32(BLK_CELLS) + lane
            for v in range(8):
                lv = lane + (16 * v)
                m = lv >= rem
                cv = cells_stage[pl.ds(o + v * 16, 16)]
                sv = scores_stage[pl.ds(o + v * 16, 16)]
                cbounce[0, pl.ds(v * 16, 16)] = jnp.where(m, dump, cv)
                sbounce[pl.ds(v * 16, 16)] = jnp.where(m, jnp.float32(0.0), sv)
            pltpu.sync_copy(sbounce, acc.at[cbounce.at[0]], add=True)
            return 0
        lax.fori_loop(0, nch, ch_body, 0)
        plsc.subcore_barrier()

        def emit_halves(nh, src_base, dst_base):
            bufs = [wbuf.at[pl.ds(0, 5000)], wbuf.at[pl.ds(5000, 5000)]]
            cpg = pltpu.async_copy(acc.at[pl.ds(src_base, 5000)], bufs[0], sem_g)
            cpg.wait()
            for h in range(nh):
                if h + 1 < nh:
                    cpn = pltpu.async_copy(
                        acc.at[pl.ds(pl.multiple_of(src_base + (h + 1) * 5000, 8), 5000)],
                        bufs[(h + 1) % 2], sem_g)
                pltpu.sync_copy(
                    bufs[h % 2],
                    out.at[pl.ds(pl.multiple_of(dst_base + h * 5000, 8), 5000)])
                if h + 1 < nh:
                    cpn.wait()

        @pl.when(b < NB - 1)
        def _():
            src0 = pl.multiple_of(tile * (4 * N_ENT), 8)
            dst0 = pl.multiple_of(
                side * (N_ENT * N_ENT) + b * BLK_CELLS + tile * (4 * N_ENT), 8)
            emit_halves(8, src0, dst0)

        @pl.when(b == NB - 1)
        def _():
            src0 = pl.multiple_of(tile * N_ENT, 8)
            dst0 = pl.multiple_of(
                side * (N_ENT * N_ENT) + (NB - 1) * BLK_CELLS + tile * N_ENT, 8)
            emit_halves(2, src0, dst0)

        plsc.subcore_barrier()

        def zr_body(k, _):
            o = pl.multiple_of(start + k * 128, 8)
            for v in range(8):
                cbounce[0, pl.ds(v * 16, 16)] = cells_stage[pl.ds(o + v * 16, 16)]
            pltpu.sync_copy(score.at[pl.ds(0, 128)], acc.at[cbounce.at[0]])
            return 0
        lax.fori_loop(0, nch, zr_body, 0)
        plsc.subcore_barrier()
        return 0

    lax.fori_loop(0, NB, blk_body, 0)


@jax.jit
def kernel(entity_emb_sr, entity_emb_tg, relation_emb_sr, relation_emb_tg,
           head_sr, tail_sr, relation_sr, head_tg, tail_tg, relation_tg):
    k = pl.kernel(
        _fused_kernel,
        out_type=jax.ShapeDtypeStruct((2 * N_ENT * N_ENT,), jnp.float32),
        mesh=_mesh,
        compiler_params=pltpu.CompilerParams(needs_layout_passes=False),
        scratch_types=[
            pltpu.VMEM_SHARED((ACC_CELLS,), jnp.float32),
            pltpu.VMEM((TPW,), jnp.int32),
            pltpu.VMEM((TPW,), jnp.int32),
            pltpu.VMEM((TPW,), jnp.int32),
            pltpu.VMEM((CHUNK, D), jnp.float32),
            pltpu.VMEM((CHUNK, D), jnp.float32),
            pltpu.VMEM((CHUNK, D), jnp.float32),
            pltpu.VMEM((TPW,), jnp.float32),
            pltpu.VMEM((REGION,), jnp.int32),
            pltpu.VMEM((REGION,), jnp.float32),
            pltpu.VMEM((NSEG,), jnp.int32),
            pltpu.VMEM((NSEG,), jnp.int32),
            pltpu.VMEM((1, 128), jnp.int32),
            pltpu.VMEM((128,), jnp.float32),
            pltpu.VMEM((N_ENT,), jnp.float32),
            pltpu.SemaphoreType.DMA,
            pltpu.SemaphoreType.DMA,
            pltpu.SemaphoreType.DMA,
            pltpu.SemaphoreType.DMA,
            pltpu.SemaphoreType.DMA,
        ],
    )
    out = k(entity_emb_sr, entity_emb_tg, relation_emb_sr, relation_emb_tg,
            head_sr, tail_sr, relation_sr, head_tg, tail_tg, relation_tg)
    return out.reshape((2, N_ENT, N_ENT))
